# baseline (speedup 1.0000x reference)
import functools

import jax
import jax.numpy as jnp
from jax import lax
from jax.experimental import pallas as pl
from jax.experimental.pallas import tpu as pltpu
from jax.experimental.pallas import tpu_sc as plsc

N = 10000
E = 320000
D = 128
NC, NS = 2, 16
NW = NC * NS
K = 128
CHUNKS = 80
EPW = K * CHUNKS
EP = EPW * NW
ACC_R = 10240
SPS = ACC_R // NS
ZR = 40
LAST = N - (NS - 1) * SPS
BR = 1000
DB = 1024


def _sc_agg_body(xe, srcp, dstp, *refs):
    (out, s0, s1, didx, rows0, rows1, zbuf, acc,
     gsem0, gsem1, isem0, isem1) = refs
    cid = lax.axis_index("c")
    sid = lax.axis_index("s")
    wid = cid * NS + sid
    row0 = sid * SPS

    @pl.loop(0, ZR * (D // 16))
    def _zb(i):
        zbuf[i // (D // 16), pl.ds((i % (D // 16)) * 16, 16)] = jnp.zeros(
            (16,), jnp.float32)

    @pl.loop(0, SPS // ZR, step=8)
    def _zg(b0):
        for j in range(8):
            pltpu.async_copy(zbuf, acc.at[pl.ds(row0 + (b0 + j) * ZR, ZR)],
                             gsem0)
        for j in range(8):
            pltpu.make_async_copy(zbuf,
                                  acc.at[pl.ds(row0 + (b0 + j) * ZR, ZR)],
                                  gsem0).wait()

    pltpu.sync_copy(dstp.at[wid], didx)

    plsc.subcore_barrier()

    base = wid * EPW

    def _sidx_start(t, buf, sem):
        pltpu.async_copy(srcp.at[pl.ds(base + t * K, K)], buf, sem)

    def _sidx_wait(t, buf, sem):
        pltpu.make_async_copy(srcp.at[pl.ds(base + t * K, K)], buf,
                              sem).wait()

    def _gather_start(buf, idxbuf, sem):
        pltpu.async_copy(xe.at[idxbuf.at[pl.ds(0, K // 2)]],
                         buf.at[pl.ds(0, K // 2)], sem)
        pltpu.async_copy(xe.at[idxbuf.at[pl.ds(K // 2, K // 2)]],
                         buf.at[pl.ds(K // 2, K // 2)], sem)

    def _gather_wait(buf, idxbuf, sem):
        pltpu.make_async_copy(xe.at[idxbuf.at[pl.ds(0, K // 2)]],
                              buf.at[pl.ds(0, K // 2)], sem).wait()
        pltpu.make_async_copy(xe.at[idxbuf.at[pl.ds(K // 2, K // 2)]],
                              buf.at[pl.ds(K // 2, K // 2)], sem).wait()

    def _scatter(t, buf):
        pltpu.sync_copy(buf, acc.at[didx.at[t]], add=True)

    pltpu.sync_copy(srcp.at[pl.ds(base, K)], s0)
    _gather_start(rows0, s0, gsem0)
    _sidx_start(1, s1, isem1)

    @pl.loop(0, CHUNKS, step=2)
    def _grp(t0):
        _sidx_wait(t0 + 1, s1, isem1)
        _gather_start(rows1, s1, gsem1)
        _gather_wait(rows0, s0, gsem0)

        @pl.when(t0 + 2 < CHUNKS)
        def _pre0():
            _sidx_start(t0 + 2, s0, isem0)

        _scatter(t0, rows0)

        @pl.when(t0 + 2 < CHUNKS)
        def _nxt():
            _sidx_wait(t0 + 2, s0, isem0)
            _gather_start(rows0, s0, gsem0)

        _gather_wait(rows1, s1, gsem1)

        @pl.when(t0 + 3 < CHUNKS)
        def _pre1():
            _sidx_start(t0 + 3, s1, isem1)

        _scatter(t0 + 1, rows1)

    plsc.subcore_barrier()

    @pl.when(sid < NS - 1)
    def _full():
        pltpu.sync_copy(acc.at[pl.ds(row0, SPS)],
                        out.at[cid, pl.ds(row0, SPS)])

    @pl.when(sid == NS - 1)
    def _short():
        pltpu.sync_copy(acc.at[pl.ds(row0, LAST)],
                        out.at[cid, pl.ds(row0, LAST)])


def _deg_hist_body(dstp2, deg_out, dbuf, cnt, sem):
    cid = lax.axis_index("c")
    sid = lax.axis_index("s")
    wid = cid * NS + sid

    @pl.loop(0, ACC_R // 16)
    def _zc(i):
        cnt[pl.ds(i * 16, 16)] = jnp.zeros((16,), jnp.float32)

    pltpu.sync_copy(dstp2.at[wid], dbuf)

    @pl.loop(0, EPW // 16)
    def _h(j):
        dv = dbuf[pl.ds(j * 16, 16)]
        plsc.addupdate_scatter(cnt, [dv], jnp.ones((16,), jnp.float32))

    pltpu.sync_copy(cnt, deg_out.at[cid, sid])


_sc_mesh = plsc.VectorSubcoreMesh(core_axis_name="c", subcore_axis_name="s")

_sc_idx_rows_scratch = [
    pltpu.VMEM((K,), jnp.int32),
    pltpu.VMEM((K,), jnp.int32),
    pltpu.VMEM((CHUNKS, K), jnp.int32),
    pltpu.VMEM((K, D), jnp.float32),
    pltpu.VMEM((K, D), jnp.float32),
    pltpu.VMEM((ZR, D), jnp.float32),
]
_sc_sems = [pltpu.SemaphoreType.DMA] * 4

_sc_agg = pl.kernel(
    _sc_agg_body,
    out_type=jax.ShapeDtypeStruct((NC, N, D), jnp.float32),
    mesh=_sc_mesh,
    scratch_types=_sc_idx_rows_scratch + [
        pltpu.VMEM_SHARED((ACC_R, D), jnp.float32),
    ] + _sc_sems,
)

_deg_hist = pl.kernel(
    _deg_hist_body,
    out_type=jax.ShapeDtypeStruct((NC, NS, ACC_R), jnp.float32),
    mesh=_sc_mesh,
    compiler_params=pltpu.CompilerParams(needs_layout_passes=False),
    scratch_types=[
        pltpu.VMEM((EPW,), jnp.int32),
        pltpu.VMEM((ACC_R,), jnp.float32),
        pltpu.SemaphoreType.DMA,
    ],
)


def _deg_format_body(dd, ident, nrm):
    s = jnp.sum(dd[...].reshape(NW, DB), axis=0, keepdims=True)
    col = lax.dot_general(ident[...], s, (((1,), (1,)), ((), ())),
                          preferred_element_type=jnp.float32)
    nrm[...] = lax.rsqrt(jnp.maximum(col, 1.0))


_deg_format = pl.pallas_call(
    _deg_format_body,
    grid=(ACC_R // DB,),
    in_specs=[
        pl.BlockSpec((NC, NS, DB), lambda i: (0, 0, i)),
        pl.BlockSpec((DB, DB), lambda i: (0, 0)),
    ],
    out_specs=pl.BlockSpec((DB, 1), lambda i: (i, 0)),
    out_shape=jax.ShapeDtypeStruct((ACC_R, 1), jnp.float32),
)


def _tc_layer_body(p0, p1, nr, xe, w, b, out, *, sig):
    agg = p0[0] + p1[0]
    h = (agg + xe[...]) * nr[...]
    y = lax.dot_general(h, w[...], (((1,), (1,)), ((), ())),
                        preferred_element_type=jnp.float32) + b[...]
    if sig:
        y = jax.nn.sigmoid(y)
    out[...] = y


def _tc_layer(p, nrm, xe, w, b, sig):
    body = functools.partial(_tc_layer_body, sig=sig)
    return pl.pallas_call(
        body,
        grid=(N // BR,),
        in_specs=[
            pl.BlockSpec((1, BR, D), lambda i: (0, i, 0)),
            pl.BlockSpec((1, BR, D), lambda i: (1, i, 0)),
            pl.BlockSpec((BR, 1), lambda i: (i, 0)),
            pl.BlockSpec((BR, D), lambda i: (i, 0)),
            pl.BlockSpec((D, D), lambda i: (0, 0)),
            pl.BlockSpec((1, D), lambda i: (0, 0)),
        ],
        out_specs=pl.BlockSpec((BR, D), lambda i: (i, 0)),
        out_shape=jax.ShapeDtypeStruct((N, D), jnp.float32),
    )(p, p, nrm, xe, w, b)


def kernel(features, edge_index, W1, b1, W2, b2):
    src = edge_index[0].astype(jnp.int32)
    dst = edge_index[1].astype(jnp.int32)
    pad = EP - E
    i = jnp.arange(pad, dtype=jnp.int32)
    srcp = jnp.concatenate([src, i % N])
    dstp = jnp.concatenate([dst, N + (i % (ACC_R - N))])
    dstp3 = dstp.reshape(NW, CHUNKS, K)
    ident = jnp.eye(DB, dtype=jnp.float32)

    deg = _deg_hist(dstp.reshape(NW, EPW))
    p1 = _sc_agg(features, srcp, dstp3)
    nrm = _deg_format(deg, ident)
    h1 = _tc_layer(p1, nrm, features, W1, b1.reshape(1, D), True)
    p2 = _sc_agg(h1, srcp, dstp3)
    return _tc_layer(p2, nrm, h1, W2, b2.reshape(1, D), False)


# speedup vs baseline: 1.0007x; 1.0007x over previous
import functools

import jax
import jax.numpy as jnp
from jax import lax
from jax.experimental import pallas as pl
from jax.experimental.pallas import tpu as pltpu
from jax.experimental.pallas import tpu_sc as plsc

N = 10000
E = 320000
D = 128
NC, NS = 2, 16
NW = NC * NS
K = 128
CHUNKS = 80
EPW = K * CHUNKS
EP = EPW * NW
ACC_R = 10240
SPS = ACC_R // NS
ZR = 40
LAST = N - (NS - 1) * SPS
BR = 1000
DB = 1024


def _sc_agg_body(xe, srcp, dstp, *refs):
    (out, s0, s1, didx, rows0, rows1, zbuf, acc,
     gsem0, gsem1, isem0, isem1) = refs
    cid = lax.axis_index("c")
    sid = lax.axis_index("s")
    wid = cid * NS + sid
    row0 = sid * SPS

    @pl.loop(0, ZR * (D // 16))
    def _zb(i):
        zbuf[i // (D // 16), pl.ds((i % (D // 16)) * 16, 16)] = jnp.zeros(
            (16,), jnp.float32)

    @pl.loop(0, SPS // ZR, step=8)
    def _zg(b0):
        for j in range(8):
            pltpu.async_copy(zbuf, acc.at[pl.ds(row0 + (b0 + j) * ZR, ZR)],
                             gsem0)
        for j in range(8):
            pltpu.make_async_copy(zbuf,
                                  acc.at[pl.ds(row0 + (b0 + j) * ZR, ZR)],
                                  gsem0).wait()

    pltpu.sync_copy(dstp.at[wid], didx)

    plsc.subcore_barrier()

    base = wid * EPW

    def _sidx_start(t, buf, sem):
        pltpu.async_copy(srcp.at[pl.ds(base + t * K, K)], buf, sem)

    def _sidx_wait(t, buf, sem):
        pltpu.make_async_copy(srcp.at[pl.ds(base + t * K, K)], buf,
                              sem).wait()

    def _gather_start(buf, idxbuf, sem):
        pltpu.async_copy(xe.at[idxbuf], buf, sem)

    def _gather_wait(buf, idxbuf, sem):
        pltpu.make_async_copy(xe.at[idxbuf], buf, sem).wait()

    def _scatter(t, buf):
        pltpu.sync_copy(buf, acc.at[didx.at[t]], add=True)

    pltpu.sync_copy(srcp.at[pl.ds(base, K)], s0)
    _gather_start(rows0, s0, gsem0)
    _sidx_start(1, s1, isem1)

    @pl.loop(0, CHUNKS, step=2)
    def _grp(t0):
        _sidx_wait(t0 + 1, s1, isem1)
        _gather_start(rows1, s1, gsem1)
        _gather_wait(rows0, s0, gsem0)

        @pl.when(t0 + 2 < CHUNKS)
        def _pre0():
            _sidx_start(t0 + 2, s0, isem0)

        _scatter(t0, rows0)

        @pl.when(t0 + 2 < CHUNKS)
        def _nxt():
            _sidx_wait(t0 + 2, s0, isem0)
            _gather_start(rows0, s0, gsem0)

        _gather_wait(rows1, s1, gsem1)

        @pl.when(t0 + 3 < CHUNKS)
        def _pre1():
            _sidx_start(t0 + 3, s1, isem1)

        _scatter(t0 + 1, rows1)

    plsc.subcore_barrier()

    @pl.when(sid < NS - 1)
    def _full():
        pltpu.sync_copy(acc.at[pl.ds(row0, SPS)],
                        out.at[cid, pl.ds(row0, SPS)])

    @pl.when(sid == NS - 1)
    def _short():
        pltpu.sync_copy(acc.at[pl.ds(row0, LAST)],
                        out.at[cid, pl.ds(row0, LAST)])


def _deg_hist_body(dstp2, deg_out, dbuf, cnt, sem):
    cid = lax.axis_index("c")
    sid = lax.axis_index("s")
    wid = cid * NS + sid

    @pl.loop(0, ACC_R // 16)
    def _zc(i):
        cnt[pl.ds(i * 16, 16)] = jnp.zeros((16,), jnp.float32)

    pltpu.sync_copy(dstp2.at[wid], dbuf)

    @pl.loop(0, EPW // 16)
    def _h(j):
        dv = dbuf[pl.ds(j * 16, 16)]
        plsc.addupdate_scatter(cnt, [dv], jnp.ones((16,), jnp.float32))

    pltpu.sync_copy(cnt, deg_out.at[cid, sid])


_sc_mesh = plsc.VectorSubcoreMesh(core_axis_name="c", subcore_axis_name="s")

_sc_idx_rows_scratch = [
    pltpu.VMEM((K,), jnp.int32),
    pltpu.VMEM((K,), jnp.int32),
    pltpu.VMEM((CHUNKS, K), jnp.int32),
    pltpu.VMEM((K, D), jnp.float32),
    pltpu.VMEM((K, D), jnp.float32),
    pltpu.VMEM((ZR, D), jnp.float32),
]
_sc_sems = [pltpu.SemaphoreType.DMA] * 4

_sc_agg = pl.kernel(
    _sc_agg_body,
    out_type=jax.ShapeDtypeStruct((NC, N, D), jnp.float32),
    mesh=_sc_mesh,
    scratch_types=_sc_idx_rows_scratch + [
        pltpu.VMEM_SHARED((ACC_R, D), jnp.float32),
    ] + _sc_sems,
)

_deg_hist = pl.kernel(
    _deg_hist_body,
    out_type=jax.ShapeDtypeStruct((NC, NS, ACC_R), jnp.float32),
    mesh=_sc_mesh,
    compiler_params=pltpu.CompilerParams(needs_layout_passes=False),
    scratch_types=[
        pltpu.VMEM((EPW,), jnp.int32),
        pltpu.VMEM((ACC_R,), jnp.float32),
        pltpu.SemaphoreType.DMA,
    ],
)


def _deg_format_body(dd, ident, nrm):
    s = jnp.sum(dd[...].reshape(NW, DB), axis=0, keepdims=True)
    col = lax.dot_general(ident[...], s, (((1,), (1,)), ((), ())),
                          preferred_element_type=jnp.float32)
    nrm[...] = lax.rsqrt(jnp.maximum(col, 1.0))


_deg_format = pl.pallas_call(
    _deg_format_body,
    grid=(ACC_R // DB,),
    in_specs=[
        pl.BlockSpec((NC, NS, DB), lambda i: (0, 0, i)),
        pl.BlockSpec((DB, DB), lambda i: (0, 0)),
    ],
    out_specs=pl.BlockSpec((DB, 1), lambda i: (i, 0)),
    out_shape=jax.ShapeDtypeStruct((ACC_R, 1), jnp.float32),
)


def _tc_layer_body(p0, p1, nr, xe, w, b, out, *, sig):
    agg = p0[0] + p1[0]
    h = (agg + xe[...]) * nr[...]
    y = lax.dot_general(h, w[...], (((1,), (1,)), ((), ())),
                        preferred_element_type=jnp.float32) + b[...]
    if sig:
        y = jax.nn.sigmoid(y)
    out[...] = y


def _tc_layer(p, nrm, xe, w, b, sig):
    body = functools.partial(_tc_layer_body, sig=sig)
    return pl.pallas_call(
        body,
        grid=(N // BR,),
        in_specs=[
            pl.BlockSpec((1, BR, D), lambda i: (0, i, 0)),
            pl.BlockSpec((1, BR, D), lambda i: (1, i, 0)),
            pl.BlockSpec((BR, 1), lambda i: (i, 0)),
            pl.BlockSpec((BR, D), lambda i: (i, 0)),
            pl.BlockSpec((D, D), lambda i: (0, 0)),
            pl.BlockSpec((1, D), lambda i: (0, 0)),
        ],
        out_specs=pl.BlockSpec((BR, D), lambda i: (i, 0)),
        out_shape=jax.ShapeDtypeStruct((N, D), jnp.float32),
    )(p, p, nrm, xe, w, b)


def kernel(features, edge_index, W1, b1, W2, b2):
    src = edge_index[0].astype(jnp.int32)
    dst = edge_index[1].astype(jnp.int32)
    pad = EP - E
    i = jnp.arange(pad, dtype=jnp.int32)
    srcp = jnp.concatenate([src, i % N])
    dstp = jnp.concatenate([dst, N + (i % (ACC_R - N))])
    dstp3 = dstp.reshape(NW, CHUNKS, K)
    ident = jnp.eye(DB, dtype=jnp.float32)

    deg = _deg_hist(dstp.reshape(NW, EPW))
    p1 = _sc_agg(features, srcp, dstp3)
    nrm = _deg_format(deg, ident)
    h1 = _tc_layer(p1, nrm, features, W1, b1.reshape(1, D), True)
    p2 = _sc_agg(h1, srcp, dstp3)
    return _tc_layer(p2, nrm, h1, W2, b2.reshape(1, D), False)
